# baseline (speedup 1.0000x reference)
import functools

import jax
import jax.numpy as jnp
from jax import lax
from jax.experimental import pallas as pl
from jax.experimental.pallas import tpu as pltpu
from jax.experimental.pallas import tpu_sc as plsc

_NC = 2
_NS = 16
_L = 16
_NW = _NC * _NS
_B = 16384
_D = 64
_BPW = _B // _NW
_CH = 128
_NCH = _BPW // _CH
_MARGIN = 1.0


def _sc_partials_body(ph, pr, pt, nh, nr, nt, ent, rel, out,
                      idx_h, idx_r, idx_t, h_v, r_v, t_v, acc_v, sem):
    wid = lax.axis_index("s") * _NC + lax.axis_index("c")
    base = wid * _BPW

    def gather_phase(ih_hbm, ir_hbm, it_hbm):
        pltpu.sync_copy(ih_hbm.at[pl.ds(base, _BPW)], idx_h)
        pltpu.sync_copy(ir_hbm.at[pl.ds(base, _BPW)], idx_r)
        pltpu.sync_copy(it_hbm.at[pl.ds(base, _BPW)], idx_t)
        copies = []
        for c in range(_NCH):
            sl = pl.ds(c * _CH, _CH)
            copies.append(pltpu.async_copy(ent.at[idx_h.at[sl]], h_v.at[sl], sem))
            copies.append(pltpu.async_copy(rel.at[idx_r.at[sl]], r_v.at[sl], sem))
            copies.append(pltpu.async_copy(ent.at[idx_t.at[sl]], t_v.at[sl], sem))
        for cp in copies:
            cp.wait()

    def l1_phase():
        def body(i, accs):
            new = []
            for j in range(_D // _L):
                sl = pl.ds(j * _L, _L)
                d = h_v[i, sl] + r_v[i, sl] - t_v[i, sl]
                new.append(accs[j] + jnp.abs(d))
            return tuple(new)

        zero = jnp.zeros((_L,), jnp.float32)
        accs = lax.fori_loop(0, _BPW, body, (zero,) * (_D // _L))
        total = accs[0]
        for a in accs[1:]:
            total = total + a
        return total

    gather_phase(ph, pr, pt)
    pos_sum = l1_phase()
    gather_phase(nh, nr, nt)
    neg_sum = l1_phase()

    acc_v[...] = neg_sum - pos_sum
    pltpu.sync_copy(acc_v, out.at[wid])


_sc_partials = functools.partial(
    pl.kernel,
    out_type=jax.ShapeDtypeStruct((_NW, _L), jnp.float32),
    mesh=plsc.VectorSubcoreMesh(
        core_axis_name="c", subcore_axis_name="s",
        num_cores=_NC, num_subcores=_NS),
    scratch_types=[
        pltpu.VMEM((_BPW,), jnp.int32),
        pltpu.VMEM((_BPW,), jnp.int32),
        pltpu.VMEM((_BPW,), jnp.int32),
        pltpu.VMEM((_BPW, _D), jnp.float32),
        pltpu.VMEM((_BPW, _D), jnp.float32),
        pltpu.VMEM((_BPW, _D), jnp.float32),
        pltpu.VMEM((_L,), jnp.float32),
        pltpu.SemaphoreType.DMA,
    ],
    compiler_params=pltpu.CompilerParams(use_tc_tiling_on_sc=False),
)(_sc_partials_body)


def _combine_body(parts_ref, out_ref):
    s = jnp.sum(parts_ref[...])
    out_ref[...] = jnp.maximum(s + _MARGIN, 0.0).reshape(1, 1)


_combine = pl.pallas_call(
    _combine_body,
    out_shape=jax.ShapeDtypeStruct((1, 1), jnp.float32),
)


@jax.jit
def kernel(pos_exmpl, neg_exmpl, entities_embeddings, relation_embeddings):
    ph, pr, pt = pos_exmpl[0], pos_exmpl[1], pos_exmpl[2]
    nh, nr, nt = neg_exmpl[0], neg_exmpl[1], neg_exmpl[2]
    parts = _sc_partials(ph, pr, pt, nh, nr, nt,
                         entities_embeddings, relation_embeddings)
    return _combine(parts)[0, 0]


# speedup vs baseline: 1.5104x; 1.5104x over previous
import functools

import jax
import jax.numpy as jnp
from jax import lax
from jax.experimental import pallas as pl
from jax.experimental.pallas import tpu as pltpu
from jax.experimental.pallas import tpu_sc as plsc

_NC = 2
_NS = 16
_L = 16
_NW = _NC * _NS
_B = 16384
_D = 64
_BPW = _B // _NW
_CH = 16
_NSTEP = _BPW // _CH
_MARGIN = 1.0


def _sc_partials_body(ph, pr, pt, nh, nr, nt, ent, rel, out,
                      idx_h, idx_r, idx_t,
                      h_buf, r_buf, t_buf, acc_v, sem0, sem1):
    wid = lax.axis_index("s") * _NC + lax.axis_index("c")
    base = wid * _BPW
    bufs = (h_buf, r_buf, t_buf)
    tbls = (ent, rel, ent)
    idxs = (idx_h, idx_r, idx_t)

    def start(k, b, sem):
        for row in range(3):
            iv = idxs[row][pl.ds(k * _CH, _CH)]
            for i in range(_CH):
                r = iv[i]
                pltpu.async_copy(
                    tbls[row].at[pl.ds(r, 1)],
                    bufs[row].at[b, pl.ds(i, 1)], sem)

    def wait(b, sem):
        for row in range(3):
            pltpu.make_async_copy(
                tbls[row].at[pl.ds(0, _CH)],
                bufs[row].at[b], sem).wait()

    def compute(b, accs):
        new = list(accs)
        for i in range(_CH):
            for j in range(_D // _L):
                sl = pl.ds(j * _L, _L)
                d = h_buf[b, i, sl] + r_buf[b, i, sl] - t_buf[b, i, sl]
                new[j] = new[j] + jnp.abs(d)
        return tuple(new)

    def run_phase(ih, ir, it):
        pltpu.sync_copy(ih.at[pl.ds(base, _BPW)], idx_h)
        pltpu.sync_copy(ir.at[pl.ds(base, _BPW)], idx_r)
        pltpu.sync_copy(it.at[pl.ds(base, _BPW)], idx_t)

        start(0, 0, sem0)

        def pair(g2, accs):
            g = g2 * 2

            @pl.when(g + 1 < _NSTEP)
            def _():
                start(g + 1, 1, sem1)
            wait(0, sem0)
            accs = compute(0, accs)

            @pl.when(g + 2 < _NSTEP)
            def _():
                start(g + 2, 0, sem0)
            wait(1, sem1)
            accs = compute(1, accs)
            return accs

        zero = jnp.zeros((_L,), jnp.float32)
        accs = lax.fori_loop(0, _NSTEP // 2, pair, (zero,) * (_D // _L))
        total = accs[0]
        for a in accs[1:]:
            total = total + a
        return total

    pos_sum = run_phase(ph, pr, pt)
    neg_sum = run_phase(nh, nr, nt)

    acc_v[...] = neg_sum - pos_sum
    pltpu.sync_copy(acc_v, out.at[wid])


_sc_partials = functools.partial(
    pl.kernel,
    out_type=jax.ShapeDtypeStruct((_NW, _L), jnp.float32),
    mesh=plsc.VectorSubcoreMesh(
        core_axis_name="c", subcore_axis_name="s",
        num_cores=_NC, num_subcores=_NS),
    scratch_types=[
        pltpu.VMEM((_BPW,), jnp.int32),
        pltpu.VMEM((_BPW,), jnp.int32),
        pltpu.VMEM((_BPW,), jnp.int32),
        pltpu.VMEM((2, _CH, _D), jnp.float32),
        pltpu.VMEM((2, _CH, _D), jnp.float32),
        pltpu.VMEM((2, _CH, _D), jnp.float32),
        pltpu.VMEM((_L,), jnp.float32),
        pltpu.SemaphoreType.DMA,
        pltpu.SemaphoreType.DMA,
    ],
)(_sc_partials_body)


def _combine_body(parts_ref, out_ref):
    s = jnp.sum(parts_ref[...])
    out_ref[...] = jnp.maximum(s + _MARGIN, 0.0).reshape(1, 1)


_combine = pl.pallas_call(
    _combine_body,
    out_shape=jax.ShapeDtypeStruct((1, 1), jnp.float32),
)


@jax.jit
def kernel(pos_exmpl, neg_exmpl, entities_embeddings, relation_embeddings):
    ph, pr, pt = pos_exmpl[0], pos_exmpl[1], pos_exmpl[2]
    nh, nr, nt = neg_exmpl[0], neg_exmpl[1], neg_exmpl[2]
    parts = _sc_partials(ph, pr, pt, nh, nr, nt,
                         entities_embeddings, relation_embeddings)
    return _combine(parts)[0, 0]
